# Initial kernel scaffold; baseline (speedup 1.0000x reference)
import jax
import jax.numpy as jnp
from jax.experimental import pallas as pl


def kernel(h_x, t_x, h_edge_index, t_edge_index, b_edge_index, h_batch, t_batch, lin0_W, lin0_b, conv_W, conv_as, conv_ad, conv_b, intra_W, intra_as, intra_ad, intra_b, inter_Ws, inter_Wd, inter_as, inter_ad, inter_b, sag_W1, sag_W2, sag_b):
    raise NotImplementedError("write your pallas kernel here")


# speedup vs baseline: 8.4785x; 8.4785x over previous
import functools

import jax
import jax.numpy as jnp
from jax import lax
from jax.experimental import pallas as pl
from jax.experimental.pallas import tpu as pltpu
from jax.experimental.pallas import tpu_sc as plsc

N = 10000
E = 320000
D = 128
H = 128
HO = 64
G = 64

NP_ = 10240
NBLK = NP_ // 128
NC = 2
NS = 16
LANES = 16

f32 = jnp.float32
i32 = jnp.int32


def _zero_fill(zbuf, zr, c):
    zv = jnp.zeros((16,), f32)
    for r in range(zr):
        for k in range(c // 16):
            zbuf[r, pl.ds(k * 16, 16)] = zv


@functools.partial(jax.jit, static_argnames=("cols",))
def _sc_routed(table, src, dst, s, d, *, cols):
    ept = E // NS
    B = 80
    nbatch = ept // B
    rpt = 2 * NP_ // NS
    ZR = 80

    def body(table_h, src_h, dst_h, s_h, d_h, out_h,
             s_v, d_v, srcb, dstb, sidx, didx, rows, acc, sem):
        c = lax.axis_index("c")
        t = lax.axis_index("s")
        pltpu.sync_copy(s_h, s_v)
        pltpu.sync_copy(d_h, d_v)
        _zero_fill(rows, ZR, cols)
        base = t * rpt
        for j in range(rpt // ZR):
            pltpu.sync_copy(rows, acc.at[pl.ds(base + j * ZR, ZR)])
        plsc.subcore_barrier()

        zero16 = jnp.zeros((16,), i32)
        npv = jnp.full((16,), NP_, i32)
        ebase = t * ept

        def batch(i, _):
            eb = ebase + i * B
            pltpu.sync_copy(src_h.at[pl.ds(eb, B)], srcb.at[0])
            pltpu.sync_copy(dst_h.at[pl.ds(eb, B)], dstb.at[0])
            for k in range(B // 16):
                sl = pl.ds(k * 16, 16)
                s16 = srcb[0, sl]
                d16 = dstb[0, sl]
                sg = plsc.load_gather(s_v, [s16])
                dg = plsc.load_gather(d_v, [d16])
                off = jnp.where(sg + dg > 0.0, zero16, npv)
                sidx[0, sl] = s16 + off + c * (2 * NP_)
                didx[0, sl] = d16 + off
            pltpu.async_copy(table_h.at[sidx.at[0]], rows, sem).wait()
            pltpu.sync_copy(rows, acc.at[didx.at[0]], add=True)
            return 0

        lax.fori_loop(0, nbatch, batch, 0)
        plsc.subcore_barrier()
        pltpu.sync_copy(acc.at[pl.ds(base, rpt)],
                        out_h.at[pl.ds(c * 2 * NP_ + base, rpt)])

    mesh = plsc.VectorSubcoreMesh(core_axis_name="c", subcore_axis_name="s")
    run = pl.kernel(
        body,
        out_type=jax.ShapeDtypeStruct((2 * 2 * NP_, cols), f32),
        mesh=mesh,
        scratch_types=[
            pltpu.VMEM((NP_,), f32),
            pltpu.VMEM((NP_,), f32),
            pltpu.VMEM((1, B), i32),
            pltpu.VMEM((1, B), i32),
            pltpu.VMEM((1, B), i32),
            pltpu.VMEM((1, B), i32),
            pltpu.VMEM((ZR, cols), f32),
            pltpu.VMEM_SHARED((2 * NP_, cols), f32),
            pltpu.SemaphoreType.DMA,
        ],
        compiler_params=pltpu.CompilerParams(needs_layout_passes=False, use_tc_tiling_on_sc=False),
    )
    return run(table, src, dst, s, d)


@functools.partial(jax.jit, static_argnames=("cols",))
def _sc_plain(table, src, dst, *, cols):
    ept = E // (NC * NS)
    B = 80
    nbatch = ept // B
    rpt = NP_ // NS
    ZR = 80

    def body(table_h, src_h, dst_h, out_h,
             src_v, dst_v, sidx, didx, rows, zbuf, acc, sem):
        c = lax.axis_index("c")
        t = lax.axis_index("s")
        w = c * NS + t
        pltpu.sync_copy(src_h.at[pl.ds(w * ept, ept)], src_v)
        pltpu.sync_copy(dst_h.at[pl.ds(w * ept, ept)], dst_v)
        _zero_fill(zbuf, ZR, cols)
        base = t * rpt
        for j in range(rpt // ZR):
            pltpu.sync_copy(zbuf, acc.at[pl.ds(base + j * ZR, ZR)])
        plsc.subcore_barrier()

        def batch(i, _):
            eb = i * B
            for k in range(B // 16):
                sl = pl.ds(eb + k * 16, 16)
                sidx[0, pl.ds(k * 16, 16)] = src_v[sl]
                didx[0, pl.ds(k * 16, 16)] = dst_v[sl]
            pltpu.async_copy(table_h.at[sidx.at[0]], rows, sem).wait()
            pltpu.sync_copy(rows, acc.at[didx.at[0]], add=True)
            return 0

        lax.fori_loop(0, nbatch, batch, 0)
        plsc.subcore_barrier()
        pltpu.sync_copy(acc.at[pl.ds(base, rpt)],
                        out_h.at[pl.ds(c * NP_ + base, rpt)])

    mesh = plsc.VectorSubcoreMesh(core_axis_name="c", subcore_axis_name="s")
    run = pl.kernel(
        body,
        out_type=jax.ShapeDtypeStruct((2 * NP_, cols), f32),
        mesh=mesh,
        scratch_types=[
            pltpu.VMEM((ept,), i32),
            pltpu.VMEM((ept,), i32),
            pltpu.VMEM((1, B), i32),
            pltpu.VMEM((1, B), i32),
            pltpu.VMEM((B, cols), f32),
            pltpu.VMEM((ZR, cols), f32),
            pltpu.VMEM_SHARED((NP_, cols), f32),
            pltpu.SemaphoreType.DMA,
        ],
        compiler_params=pltpu.CompilerParams(needs_layout_passes=False, use_tc_tiling_on_sc=False),
    )
    return run(table, src, dst)


def _full(shape):
    return pl.BlockSpec(shape, lambda i: (0,) * len(shape))


def _prep(x, W0, b0, Wc, A):
    def body(x_r, w0_r, b0_r, wc_r, a_r, xp_r, sd_r, bm_r):
        y = jnp.dot(x_r[...], w0_r[...], preferred_element_type=f32) + b0_r[...]
        xp = jnp.dot(y, wc_r[...], preferred_element_type=f32)
        sd = jnp.dot(xp, a_r[...], preferred_element_type=f32)
        xp_r[...] = xp
        sd_r[...] = sd
        bm_r[...] = jnp.max(sd, axis=0, keepdims=True)[None]

    return pl.pallas_call(
        body,
        grid=(NBLK,),
        in_specs=[
            pl.BlockSpec((128, D), lambda i: (i, 0)),
            _full((D, H)), _full((1, H)), _full((H, H)), _full((H, 2)),
        ],
        out_specs=[
            pl.BlockSpec((128, H), lambda i: (i, 0)),
            pl.BlockSpec((128, 2), lambda i: (i, 0)),
            pl.BlockSpec((1, 1, 2), lambda i: (i, 0, 0)),
        ],
        out_shape=[
            jax.ShapeDtypeStruct((NP_, H), f32),
            jax.ShapeDtypeStruct((NP_, 2), f32),
            jax.ShapeDtypeStruct((NBLK, 1, 2), f32),
        ],
    )(x, W0, b0, Wc, A)


def _proj(x, W, A, act):
    M = W.shape[1]

    def body(x_r, w_r, a_r, xp_r, sd_r, bm_r):
        xv = x_r[...]
        if act == "elu":
            xv = jnp.where(xv > 0, xv, jnp.exp(jnp.minimum(xv, 0.0)) - 1.0)
        xp = jnp.dot(xv, w_r[...], preferred_element_type=f32)
        sd = jnp.dot(xp, a_r[...], preferred_element_type=f32)
        xp_r[...] = xp
        sd_r[...] = sd
        bm_r[...] = jnp.max(sd, axis=0, keepdims=True)[None]

    return pl.pallas_call(
        body,
        grid=(NBLK,),
        in_specs=[
            pl.BlockSpec((128, x.shape[1]), lambda i: (i, 0)),
            _full((x.shape[1], M)), _full((M, 2)),
        ],
        out_specs=[
            pl.BlockSpec((128, M), lambda i: (i, 0)),
            pl.BlockSpec((128, 2), lambda i: (i, 0)),
            pl.BlockSpec((1, 1, 2), lambda i: (i, 0, 0)),
        ],
        out_shape=[
            jax.ShapeDtypeStruct((NP_, M), f32),
            jax.ShapeDtypeStruct((NP_, 2), f32),
            jax.ShapeDtypeStruct((NBLK, 1, 2), f32),
        ],
    )(x, W, A)


def _tables(xp, sd, K, cf, cols):
    ch = cf // 2

    def body(xp_r, sd_r, k_r, tab_r, dd_r):
        s = sd_r[...][:, 0:1]
        d = sd_r[...][:, 1:2]
        kv = k_r[...]
        p = jnp.exp(s - kv[0, 0])
        n = jnp.exp(0.2 * s - kv[0, 1])
        dd_r[...] = jnp.concatenate(
            [jnp.exp(d - kv[0, 2]), jnp.exp(0.2 * d - kv[0, 3])], axis=1)
        xpv = xp_r[...]
        pad = jnp.zeros((128, cols - ch - 1), f32)
        for c in range(2):
            xc = xpv[:, c * ch:(c + 1) * ch]
            tab_r[c, 0, :, :] = jnp.concatenate([xc * p, p, pad], axis=1)
            tab_r[c, 1, :, :] = jnp.concatenate([xc * n, n, pad], axis=1)

    return pl.pallas_call(
        body,
        grid=(NBLK,),
        in_specs=[
            pl.BlockSpec((128, cf), lambda i: (i, 0)),
            pl.BlockSpec((128, 2), lambda i: (i, 0)),
            _full((1, 4)),
        ],
        out_specs=[
            pl.BlockSpec((2, 2, 128, cols), lambda i: (0, 0, i, 0)),
            pl.BlockSpec((128, 2), lambda i: (i, 0)),
        ],
        out_shape=[
            jax.ShapeDtypeStruct((2, 2, NP_, cols), f32),
            jax.ShapeDtypeStruct((NP_, 2), f32),
        ],
    )(xp, sd, K)


def _combine(accd, dd, bias, cf, cols, act):
    ch = cf // 2

    def body(a0_r, a1_r, a2_r, a3_r, dd_r, b_r, y_r):
        dP = dd_r[...][:, 0:1]
        dN = dd_r[...][:, 1:2]
        halves = []
        for (ap, an) in ((a0_r, a1_r), (a2_r, a3_r)):
            apv = ap[...]
            anv = an[...]
            num = dP * apv[:, :ch] + dN * anv[:, :ch]
            den = dP * apv[:, ch:ch + 1] + dN * anv[:, ch:ch + 1] + 1e-16
            halves.append(num / den)
        y = jnp.concatenate(halves, axis=1) + b_r[...]
        if act == "relu":
            y = jnp.maximum(y, 0.0)
        y_r[...] = y

    specs = [pl.BlockSpec((128, cols), lambda i, q=q: (q * NBLK + i, 0))
             for q in range(4)]
    return pl.pallas_call(
        body,
        grid=(NBLK,),
        in_specs=specs + [
            pl.BlockSpec((128, 2), lambda i: (i, 0)),
            _full((1, cf)),
        ],
        out_specs=pl.BlockSpec((128, cf), lambda i: (i, 0)),
        out_shape=jax.ShapeDtypeStruct((NP_, cf), f32),
    )(accd, accd, accd, accd, dd, bias)


def _sag_prep(rep, W12):
    def body(x_r, w_r, sg_r, tab_r):
        sg = jnp.dot(x_r[...], w_r[...], preferred_element_type=f32)
        sg_r[...] = sg
        tab_r[...] = jnp.concatenate(
            [sg[:, 1:2], jnp.zeros((128, 15), f32)], axis=1)

    return pl.pallas_call(
        body,
        grid=(NBLK,),
        in_specs=[pl.BlockSpec((128, H), lambda i: (i, 0)), _full((H, 2))],
        out_specs=[
            pl.BlockSpec((128, 2), lambda i: (i, 0)),
            pl.BlockSpec((128, 16), lambda i: (i, 0)),
        ],
        out_shape=[
            jax.ShapeDtypeStruct((NP_, 2), f32),
            jax.ShapeDtypeStruct((NP_, 16), f32),
        ],
    )(rep, W12)


def _pooling(rep, sg, aggd, batch2, sb):
    def body(rep_r, sg_r, agg_r, b2_r, sb_r, emb_r, wm_r):
        aggv = agg_r[...][:NP_, 0] + agg_r[...][NP_:, 0]
        sraw = sg_r[...][:, 0] + aggv + sb_r[0, 0]
        s2 = sraw.reshape(NBLK, 128)
        b2 = b2_r[...]
        neg = jnp.float32(-1e30)

        def mloop(g, mpick):
            mask = b2 == g
            mg = jnp.max(jnp.where(mask, s2, neg))
            mg = jnp.where(mg > -1e29, mg, 0.0)
            return mpick + jnp.where(mask, mg, 0.0)

        mpick = lax.fori_loop(0, G, mloop, jnp.zeros((NBLK, 128), f32))
        e2 = jnp.exp(s2 - mpick)

        def zloop(g, zpick):
            mask = b2 == g
            zg = jnp.sum(jnp.where(mask, e2, 0.0))
            return zpick + jnp.where(mask, zg, 0.0)

        zpick = lax.fori_loop(0, G, zloop, jnp.zeros((NBLK, 128), f32))
        score2 = e2 / (zpick + 1e-16)
        sflat = score2.reshape(1, NP_)
        bflat = b2.reshape(1, NP_)

        def wloop(g, _):
            wm_r[pl.ds(g, 1), :] = jnp.where(bflat == g, sflat, 0.0)
            return 0

        lax.fori_loop(0, G, wloop, 0)
        emb_r[...] = jnp.dot(wm_r[...], rep_r[...],
                             preferred_element_type=f32)

    def _f0(shape):
        return pl.BlockSpec(shape, lambda: (0,) * len(shape))

    return pl.pallas_call(
        body,
        in_specs=[
            _f0((NP_, H)), _f0((NP_, 2)), _f0((2 * NP_, 16)),
            _f0((NBLK, 128)), _f0((1, 1)),
        ],
        out_specs=_f0((G, H)),
        out_shape=jax.ShapeDtypeStruct((G, H), f32),
        scratch_shapes=[pltpu.VMEM((G, NP_), f32)],
    )(rep, sg, aggd, batch2, sb)


def _shifts(bmax_s, bmax_d):
    ks = jnp.max(bmax_s)
    kd = jnp.max(bmax_d)
    bc = 0.4 * (ks + kd)
    return jnp.stack([ks, 0.2 * ks + bc, kd, 0.2 * kd + bc]).reshape(1, 4)


def _pad_rows(x):
    return jnp.pad(x, ((0, NP_ - N), (0, 0)))


def kernel(h_x, t_x, h_edge_index, t_edge_index, b_edge_index, h_batch,
           t_batch, lin0_W, lin0_b, conv_W, conv_as, conv_ad, conv_b,
           intra_W, intra_as, intra_ad, intra_b, inter_Ws, inter_Wd,
           inter_as, inter_ad, inter_b, sag_W1, sag_W2, sag_b):
    hx = _pad_rows(h_x)
    tx = _pad_rows(t_x)
    h_src = h_edge_index[0].astype(i32)
    h_dst = h_edge_index[1].astype(i32)
    t_src = t_edge_index[0].astype(i32)
    t_dst = t_edge_index[1].astype(i32)
    b_src = b_edge_index[0].astype(i32)
    b_dst = b_edge_index[1].astype(i32)
    b0 = lin0_b.reshape(1, H)

    A_conv = jnp.stack([conv_as, conv_ad], axis=1)
    bias_conv = conv_b.reshape(1, H)

    def conv_stage(x, src, dst):
        xp, sd, bm = _prep(x, lin0_W, b0, conv_W, A_conv)
        K = _shifts(bm[:, 0, 0], bm[:, 0, 1])
        tab, dd = _tables(xp, sd, K, H, 80)
        accd = _sc_routed(tab.reshape(4 * NP_, 80), src, dst,
                          sd[:, 0], sd[:, 1], cols=80)
        return _combine(accd, dd, bias_conv, H, 80, "relu")

    hx1 = conv_stage(hx, h_src, h_dst)
    tx1 = conv_stage(tx, t_src, t_dst)

    A_intra = jnp.stack([intra_as, intra_ad], axis=1)
    bias_intra = intra_b.reshape(1, HO)

    def intra_stage(x1, src, dst):
        xp, sd, bm = _proj(x1, intra_W, A_intra, "elu")
        K = _shifts(bm[:, 0, 0], bm[:, 0, 1])
        tab, dd = _tables(xp, sd, K, HO, 48)
        accd = _sc_routed(tab.reshape(4 * NP_, 48), src, dst,
                          sd[:, 0], sd[:, 1], cols=48)
        return _combine(accd, dd, bias_intra, HO, 48, "none")

    h_intra = intra_stage(hx1, h_src, h_dst)
    t_intra = intra_stage(tx1, t_src, t_dst)

    As1 = jnp.stack([inter_as, inter_as], axis=1)
    Ad1 = jnp.stack([inter_ad, inter_ad], axis=1)
    bias_inter = inter_b.reshape(1, HO)

    xps_h, sds_h, bms_h = _proj(hx1, inter_Ws, As1, "elu")
    xpd_h, sdd_h, bmd_h = _proj(hx1, inter_Wd, Ad1, "elu")
    xps_t, sds_t, bms_t = _proj(tx1, inter_Ws, As1, "elu")
    xpd_t, sdd_t, bmd_t = _proj(tx1, inter_Wd, Ad1, "elu")

    def inter_stage(xps, s_vec, bm_s, d_vec, bm_d, src, dst):
        K = _shifts(bm_s, bm_d)
        sd = jnp.stack([s_vec, d_vec], axis=1)
        tab, dd = _tables(xps, sd, K, HO, 48)
        accd = _sc_routed(tab.reshape(4 * NP_, 48), src, dst,
                          s_vec, d_vec, cols=48)
        return _combine(accd, dd, bias_inter, HO, 48, "none")

    t_inter = inter_stage(xps_h, sds_h[:, 0], bms_h[:, 0, 0],
                          sdd_t[:, 1], bmd_t[:, 0, 1], b_src, b_dst)
    h_inter = inter_stage(xps_t, sds_t[:, 0], bms_t[:, 0, 0],
                          sdd_h[:, 1], bmd_h[:, 0, 1], b_dst, b_src)

    h_rep_p = jnp.concatenate([h_intra, h_inter], axis=1)
    t_rep_p = jnp.concatenate([t_intra, t_inter], axis=1)

    W12 = jnp.concatenate([sag_W1, sag_W2], axis=1)
    sbv = sag_b.reshape(1, 1)

    def pool_stage(rep_p, src, dst, batch):
        sg, tab16 = _sag_prep(rep_p, W12)
        aggd = _sc_plain(tab16, src, dst, cols=16)
        b2 = jnp.pad(batch.astype(i32), (0, NP_ - N),
                     constant_values=G).reshape(NBLK, 128)
        return _pooling(rep_p, sg, aggd, b2, sbv)

    h_emb = pool_stage(h_rep_p, h_src, h_dst, h_batch)
    t_emb = pool_stage(t_rep_p, t_src, t_dst, t_batch)

    return (h_rep_p[:N], t_rep_p[:N], h_emb, t_emb)


# speedup vs baseline: 14.2581x; 1.6817x over previous
import functools

import jax
import jax.numpy as jnp
from jax import lax
from jax.experimental import pallas as pl
from jax.experimental.pallas import tpu as pltpu
from jax.experimental.pallas import tpu_sc as plsc

N = 10000
E = 320000
D = 128
H = 128
HO = 64
G = 64

NP_ = 10240
NBLK = NP_ // 128
NC = 2
NS = 16
LANES = 16

f32 = jnp.float32
i32 = jnp.int32


def _zero_fill(zbuf, zr, c):
    zv = jnp.zeros((16,), f32)
    for r in range(zr):
        for k in range(c // 16):
            zbuf[r, pl.ds(k * 16, 16)] = zv


@jax.jit
def _sc_route(src, dst, s, d):
    ept = E // (NC * NS)

    def body(src_h, dst_h, s_h, d_h, sr_h, dr_h,
             s_v, d_v, src_v, dst_v, sro, dro):
        c = lax.axis_index("c")
        t = lax.axis_index("s")
        w = c * NS + t
        pltpu.sync_copy(s_h, s_v)
        pltpu.sync_copy(d_h, d_v)
        pltpu.sync_copy(src_h.at[pl.ds(w * ept, ept)], src_v)
        pltpu.sync_copy(dst_h.at[pl.ds(w * ept, ept)], dst_v)
        zero16 = jnp.zeros((16,), i32)
        npv = jnp.full((16,), NP_, i32)

        def it(k, _):
            sl = pl.ds(k * 16, 16)
            s16 = src_v[sl]
            d16 = dst_v[sl]
            sg = plsc.load_gather(s_v, [s16])
            dg = plsc.load_gather(d_v, [d16])
            off = jnp.where(sg + dg > 0.0, zero16, npv)
            sro[sl] = s16 + off
            dro[sl] = d16 + off
            return 0

        lax.fori_loop(0, ept // 16, it, 0)
        pltpu.sync_copy(sro, sr_h.at[pl.ds(w * ept, ept)])
        pltpu.sync_copy(dro, dr_h.at[pl.ds(w * ept, ept)])

    mesh = plsc.VectorSubcoreMesh(core_axis_name="c", subcore_axis_name="s")
    run = pl.kernel(
        body,
        out_type=[jax.ShapeDtypeStruct((E,), i32),
                  jax.ShapeDtypeStruct((E,), i32)],
        mesh=mesh,
        scratch_types=[
            pltpu.VMEM((NP_,), f32),
            pltpu.VMEM((NP_,), f32),
            pltpu.VMEM((ept,), i32),
            pltpu.VMEM((ept,), i32),
            pltpu.VMEM((ept,), i32),
            pltpu.VMEM((ept,), i32),
        ],
        compiler_params=pltpu.CompilerParams(needs_layout_passes=False, use_tc_tiling_on_sc=False),
    )
    return run(src, dst, s, d)


@functools.partial(jax.jit, static_argnames=("cols", "routed"))
def _sc_rows(table3, sidx2, didx2, *, cols, routed):
    B = 80
    NBUF = 3
    CH = 25
    arows = (2 * NP_) if routed else NP_
    ept = E // NS if routed else E // (NC * NS)
    nbatch = ept // B
    nchunk = nbatch // CH
    rpt = arows // NS
    ZR = 80

    def body(table_h, sidx_h, didx_h, out_h,
             sic, dic, r0, r1, r2, acc, gs0, gs1, gs2, ss0, ss1, ss2):
        c = lax.axis_index("c")
        t = lax.axis_index("s")
        rows_b = (r0, r1, r2)
        gsem = (gs0, gs1, gs2)
        ssem = (ss0, ss1, ss2)
        tab = table_h.at[c] if routed else table_h.at[0]
        w = t if routed else c * NS + t
        _zero_fill(r0, ZR, cols)
        base = t * rpt
        for j in range(rpt // ZR):
            pltpu.sync_copy(r0, acc.at[pl.ds(base + j * ZR, ZR)])
        plsc.subcore_barrier()

        brow0 = w * nbatch
        for ch in range(nchunk):
            pltpu.sync_copy(sidx_h.at[pl.ds(brow0 + ch * CH, CH)], sic)
            pltpu.sync_copy(didx_h.at[pl.ds(brow0 + ch * CH, CH)], dic)
            pltpu.async_copy(tab.at[sic.at[0]], r0, gs0)
            pltpu.async_copy(tab.at[sic.at[1]], r1, gs1)
            for j in range(CH):
                r = j % NBUF
                jn = j + 2
                if jn < CH:
                    rn = jn % NBUF
                    if j >= 1:
                        pltpu.make_async_copy(rows_b[(j - 1) % NBUF],
                                              acc.at[dic.at[j - 1]],
                                              ssem[(j - 1) % NBUF]).wait()
                    pltpu.async_copy(tab.at[sic.at[jn]], rows_b[rn], gsem[rn])
                pltpu.make_async_copy(tab.at[sic.at[j]], rows_b[r],
                                      gsem[r]).wait()
                pltpu.async_copy(rows_b[r], acc.at[dic.at[j]], ssem[r],
                                 add=True)
            for j in (CH - 3, CH - 2, CH - 1):
                pltpu.make_async_copy(rows_b[j % NBUF],
                                      acc.at[dic.at[j]],
                                      ssem[j % NBUF]).wait()
        plsc.subcore_barrier()
        pltpu.sync_copy(acc.at[pl.ds(base, rpt)],
                        out_h.at[pl.ds(c * arows + base, rpt)])

    mesh = plsc.VectorSubcoreMesh(core_axis_name="c", subcore_axis_name="s")
    run = pl.kernel(
        body,
        out_type=jax.ShapeDtypeStruct((2 * arows, cols), f32),
        mesh=mesh,
        scratch_types=[
            pltpu.VMEM((CH, B), i32),
            pltpu.VMEM((CH, B), i32),
            pltpu.VMEM((B, cols), f32),
            pltpu.VMEM((B, cols), f32),
            pltpu.VMEM((B, cols), f32),
            pltpu.VMEM_SHARED((arows, cols), f32),
            pltpu.SemaphoreType.DMA,
            pltpu.SemaphoreType.DMA,
            pltpu.SemaphoreType.DMA,
            pltpu.SemaphoreType.DMA,
            pltpu.SemaphoreType.DMA,
            pltpu.SemaphoreType.DMA,
        ],
        compiler_params=pltpu.CompilerParams(needs_layout_passes=False, use_tc_tiling_on_sc=False),
    )
    return run(table3, sidx2, didx2)


def _full(shape):
    return pl.BlockSpec(shape, lambda i: (0,) * len(shape))


def _prep(x, W0, b0, Wc, A):
    def body(x_r, w0_r, b0_r, wc_r, a_r, xp_r, sd_r, bm_r):
        y = jnp.dot(x_r[...], w0_r[...], preferred_element_type=f32) + b0_r[...]
        xp = jnp.dot(y, wc_r[...], preferred_element_type=f32)
        sd = jnp.dot(xp, a_r[...], preferred_element_type=f32)
        xp_r[...] = xp
        sd_r[...] = sd
        bm_r[...] = jnp.max(sd, axis=0, keepdims=True)[None]

    return pl.pallas_call(
        body,
        grid=(NBLK,),
        in_specs=[
            pl.BlockSpec((128, D), lambda i: (i, 0)),
            _full((D, H)), _full((1, H)), _full((H, H)), _full((H, 2)),
        ],
        out_specs=[
            pl.BlockSpec((128, H), lambda i: (i, 0)),
            pl.BlockSpec((128, 2), lambda i: (i, 0)),
            pl.BlockSpec((1, 1, 2), lambda i: (i, 0, 0)),
        ],
        out_shape=[
            jax.ShapeDtypeStruct((NP_, H), f32),
            jax.ShapeDtypeStruct((NP_, 2), f32),
            jax.ShapeDtypeStruct((NBLK, 1, 2), f32),
        ],
    )(x, W0, b0, Wc, A)


def _proj(x, W, A, act):
    M = W.shape[1]

    def body(x_r, w_r, a_r, xp_r, sd_r, bm_r):
        xv = x_r[...]
        if act == "elu":
            xv = jnp.where(xv > 0, xv, jnp.exp(jnp.minimum(xv, 0.0)) - 1.0)
        xp = jnp.dot(xv, w_r[...], preferred_element_type=f32)
        sd = jnp.dot(xp, a_r[...], preferred_element_type=f32)
        xp_r[...] = xp
        sd_r[...] = sd
        bm_r[...] = jnp.max(sd, axis=0, keepdims=True)[None]

    return pl.pallas_call(
        body,
        grid=(NBLK,),
        in_specs=[
            pl.BlockSpec((128, x.shape[1]), lambda i: (i, 0)),
            _full((x.shape[1], M)), _full((M, 2)),
        ],
        out_specs=[
            pl.BlockSpec((128, M), lambda i: (i, 0)),
            pl.BlockSpec((128, 2), lambda i: (i, 0)),
            pl.BlockSpec((1, 1, 2), lambda i: (i, 0, 0)),
        ],
        out_shape=[
            jax.ShapeDtypeStruct((NP_, M), f32),
            jax.ShapeDtypeStruct((NP_, 2), f32),
            jax.ShapeDtypeStruct((NBLK, 1, 2), f32),
        ],
    )(x, W, A)


def _tables(xp, sd, K, cf, cols):
    ch = cf // 2

    def body(xp_r, sd_r, k_r, tab_r, dd_r):
        s = sd_r[...][:, 0:1]
        d = sd_r[...][:, 1:2]
        kv = k_r[...]
        p = jnp.exp(s - kv[0, 0])
        n = jnp.exp(0.2 * s - kv[0, 1])
        dd_r[...] = jnp.concatenate(
            [jnp.exp(d - kv[0, 2]), jnp.exp(0.2 * d - kv[0, 3])], axis=1)
        xpv = xp_r[...]
        pad = jnp.zeros((128, cols - ch - 1), f32)
        for c in range(2):
            xc = xpv[:, c * ch:(c + 1) * ch]
            tab_r[c, 0, :, :] = jnp.concatenate([xc * p, p, pad], axis=1)
            tab_r[c, 1, :, :] = jnp.concatenate([xc * n, n, pad], axis=1)

    return pl.pallas_call(
        body,
        grid=(NBLK,),
        in_specs=[
            pl.BlockSpec((128, cf), lambda i: (i, 0)),
            pl.BlockSpec((128, 2), lambda i: (i, 0)),
            _full((1, 4)),
        ],
        out_specs=[
            pl.BlockSpec((2, 2, 128, cols), lambda i: (0, 0, i, 0)),
            pl.BlockSpec((128, 2), lambda i: (i, 0)),
        ],
        out_shape=[
            jax.ShapeDtypeStruct((2, 2, NP_, cols), f32),
            jax.ShapeDtypeStruct((NP_, 2), f32),
        ],
    )(xp, sd, K)


def _combine(accd, dd, bias, cf, cols, act):
    ch = cf // 2

    def body(a0_r, a1_r, a2_r, a3_r, dd_r, b_r, y_r):
        dP = dd_r[...][:, 0:1]
        dN = dd_r[...][:, 1:2]
        halves = []
        for (ap, an) in ((a0_r, a1_r), (a2_r, a3_r)):
            apv = ap[...]
            anv = an[...]
            num = dP * apv[:, :ch] + dN * anv[:, :ch]
            den = dP * apv[:, ch:ch + 1] + dN * anv[:, ch:ch + 1] + 1e-16
            halves.append(num / den)
        y = jnp.concatenate(halves, axis=1) + b_r[...]
        if act == "relu":
            y = jnp.maximum(y, 0.0)
        y_r[...] = y

    specs = [pl.BlockSpec((128, cols), lambda i, q=q: (q * NBLK + i, 0))
             for q in range(4)]
    return pl.pallas_call(
        body,
        grid=(NBLK,),
        in_specs=specs + [
            pl.BlockSpec((128, 2), lambda i: (i, 0)),
            _full((1, cf)),
        ],
        out_specs=pl.BlockSpec((128, cf), lambda i: (i, 0)),
        out_shape=jax.ShapeDtypeStruct((NP_, cf), f32),
    )(accd, accd, accd, accd, dd, bias)


def _sag_prep(rep, W12):
    def body(x_r, w_r, sg_r, tab_r):
        sg = jnp.dot(x_r[...], w_r[...], preferred_element_type=f32)
        sg_r[...] = sg
        tab_r[...] = jnp.concatenate(
            [sg[:, 1:2], jnp.zeros((128, 15), f32)], axis=1)

    return pl.pallas_call(
        body,
        grid=(NBLK,),
        in_specs=[pl.BlockSpec((128, H), lambda i: (i, 0)), _full((H, 2))],
        out_specs=[
            pl.BlockSpec((128, 2), lambda i: (i, 0)),
            pl.BlockSpec((128, 16), lambda i: (i, 0)),
        ],
        out_shape=[
            jax.ShapeDtypeStruct((NP_, 2), f32),
            jax.ShapeDtypeStruct((NP_, 16), f32),
        ],
    )(rep, W12)


def _pooling(rep, sg, aggd, batch2, sb):
    def body(rep_r, sg_r, agg_r, b2_r, sb_r, emb_r, wm_r):
        aggv = agg_r[...][:NP_, 0] + agg_r[...][NP_:, 0]
        sraw = sg_r[...][:, 0] + aggv + sb_r[0, 0]
        s2 = sraw.reshape(NBLK, 128)
        b2 = b2_r[...]
        neg = jnp.float32(-1e30)

        def mloop(g, mpick):
            mask = b2 == g
            mg = jnp.max(jnp.where(mask, s2, neg))
            mg = jnp.where(mg > -1e29, mg, 0.0)
            return mpick + jnp.where(mask, mg, 0.0)

        mpick = lax.fori_loop(0, G, mloop, jnp.zeros((NBLK, 128), f32))
        e2 = jnp.exp(s2 - mpick)

        def zloop(g, zpick):
            mask = b2 == g
            zg = jnp.sum(jnp.where(mask, e2, 0.0))
            return zpick + jnp.where(mask, zg, 0.0)

        zpick = lax.fori_loop(0, G, zloop, jnp.zeros((NBLK, 128), f32))
        score2 = e2 / (zpick + 1e-16)
        sflat = score2.reshape(1, NP_)
        bflat = b2.reshape(1, NP_)

        def wloop(g, _):
            wm_r[pl.ds(g, 1), :] = jnp.where(bflat == g, sflat, 0.0)
            return 0

        lax.fori_loop(0, G, wloop, 0)
        emb_r[...] = jnp.dot(wm_r[...], rep_r[...],
                             preferred_element_type=f32)

    def _f0(shape):
        return pl.BlockSpec(shape, lambda: (0,) * len(shape))

    return pl.pallas_call(
        body,
        in_specs=[
            _f0((NP_, H)), _f0((NP_, 2)), _f0((2 * NP_, 16)),
            _f0((NBLK, 128)), _f0((1, 1)),
        ],
        out_specs=_f0((G, H)),
        out_shape=jax.ShapeDtypeStruct((G, H), f32),
        scratch_shapes=[pltpu.VMEM((G, NP_), f32)],
    )(rep, sg, aggd, batch2, sb)


def _shifts(bmax_s, bmax_d):
    ks = jnp.max(bmax_s)
    kd = jnp.max(bmax_d)
    bc = 0.4 * (ks + kd)
    return jnp.stack([ks, 0.2 * ks + bc, kd, 0.2 * kd + bc]).reshape(1, 4)


def _pad_rows(x):
    return jnp.pad(x, ((0, NP_ - N), (0, 0)))


def kernel(h_x, t_x, h_edge_index, t_edge_index, b_edge_index, h_batch,
           t_batch, lin0_W, lin0_b, conv_W, conv_as, conv_ad, conv_b,
           intra_W, intra_as, intra_ad, intra_b, inter_Ws, inter_Wd,
           inter_as, inter_ad, inter_b, sag_W1, sag_W2, sag_b):
    hx = _pad_rows(h_x)
    tx = _pad_rows(t_x)
    h_src = h_edge_index[0].astype(i32)
    h_dst = h_edge_index[1].astype(i32)
    t_src = t_edge_index[0].astype(i32)
    t_dst = t_edge_index[1].astype(i32)
    b_src = b_edge_index[0].astype(i32)
    b_dst = b_edge_index[1].astype(i32)
    b0 = lin0_b.reshape(1, H)

    A_conv = jnp.stack([conv_as, conv_ad], axis=1)
    bias_conv = conv_b.reshape(1, H)

    def conv_stage(x, src, dst):
        xp, sd, bm = _prep(x, lin0_W, b0, conv_W, A_conv)
        K = _shifts(bm[:, 0, 0], bm[:, 0, 1])
        tab, dd = _tables(xp, sd, K, H, 80)
        sr, dr = _sc_route(src, dst, sd[:, 0], sd[:, 1])
        accd = _sc_rows(tab.reshape(2, 2 * NP_, 80), sr.reshape(E // 80, 80),
                        dr.reshape(E // 80, 80), cols=80, routed=True)
        return _combine(accd, dd, bias_conv, H, 80, "relu")

    hx1 = conv_stage(hx, h_src, h_dst)
    tx1 = conv_stage(tx, t_src, t_dst)

    A_intra = jnp.stack([intra_as, intra_ad], axis=1)
    bias_intra = intra_b.reshape(1, HO)

    def intra_stage(x1, src, dst):
        xp, sd, bm = _proj(x1, intra_W, A_intra, "elu")
        K = _shifts(bm[:, 0, 0], bm[:, 0, 1])
        tab, dd = _tables(xp, sd, K, HO, 48)
        sr, dr = _sc_route(src, dst, sd[:, 0], sd[:, 1])
        accd = _sc_rows(tab.reshape(2, 2 * NP_, 48), sr.reshape(E // 80, 80),
                        dr.reshape(E // 80, 80), cols=48, routed=True)
        return _combine(accd, dd, bias_intra, HO, 48, "none")

    h_intra = intra_stage(hx1, h_src, h_dst)
    t_intra = intra_stage(tx1, t_src, t_dst)

    As1 = jnp.stack([inter_as, inter_as], axis=1)
    Ad1 = jnp.stack([inter_ad, inter_ad], axis=1)
    bias_inter = inter_b.reshape(1, HO)

    xps_h, sds_h, bms_h = _proj(hx1, inter_Ws, As1, "elu")
    xpd_h, sdd_h, bmd_h = _proj(hx1, inter_Wd, Ad1, "elu")
    xps_t, sds_t, bms_t = _proj(tx1, inter_Ws, As1, "elu")
    xpd_t, sdd_t, bmd_t = _proj(tx1, inter_Wd, Ad1, "elu")

    def inter_stage(xps, s_vec, bm_s, d_vec, bm_d, src, dst):
        K = _shifts(bm_s, bm_d)
        sd = jnp.stack([s_vec, d_vec], axis=1)
        tab, dd = _tables(xps, sd, K, HO, 48)
        sr, dr = _sc_route(src, dst, s_vec, d_vec)
        accd = _sc_rows(tab.reshape(2, 2 * NP_, 48), sr.reshape(E // 80, 80),
                        dr.reshape(E // 80, 80), cols=48, routed=True)
        return _combine(accd, dd, bias_inter, HO, 48, "none")

    t_inter = inter_stage(xps_h, sds_h[:, 0], bms_h[:, 0, 0],
                          sdd_t[:, 1], bmd_t[:, 0, 1], b_src, b_dst)
    h_inter = inter_stage(xps_t, sds_t[:, 0], bms_t[:, 0, 0],
                          sdd_h[:, 1], bmd_h[:, 0, 1], b_dst, b_src)

    h_rep_p = jnp.concatenate([h_intra, h_inter], axis=1)
    t_rep_p = jnp.concatenate([t_intra, t_inter], axis=1)

    W12 = jnp.concatenate([sag_W1, sag_W2], axis=1)
    sbv = sag_b.reshape(1, 1)

    def pool_stage(rep_p, src, dst, batch):
        sg, tab16 = _sag_prep(rep_p, W12)
        aggd = _sc_rows(tab16[None], src.reshape(E // 80, 80),
                        dst.reshape(E // 80, 80), cols=16, routed=False)
        b2 = jnp.pad(batch.astype(i32), (0, NP_ - N),
                     constant_values=G).reshape(NBLK, 128)
        return _pooling(rep_p, sg, aggd, b2, sbv)

    h_emb = pool_stage(h_rep_p, h_src, h_dst, h_batch)
    t_emb = pool_stage(t_rep_p, t_src, t_dst, t_batch)

    return (h_rep_p[:N], t_rep_p[:N], h_emb, t_emb)


# speedup vs baseline: 15.6461x; 1.0973x over previous
import functools

import jax
import jax.numpy as jnp
from jax import lax
from jax.experimental import pallas as pl
from jax.experimental.pallas import tpu as pltpu
from jax.experimental.pallas import tpu_sc as plsc

N = 10000
E = 320000
D = 128
H = 128
HO = 64
G = 64

NP_ = 10240
NBLK = NP_ // 128
NC = 2
NS = 16
LANES = 16

f32 = jnp.float32
i32 = jnp.int32


def _zero_fill(zbuf, zr, c):
    zv = jnp.zeros((16,), f32)
    for r in range(zr):
        for k in range(c // 16):
            zbuf[r, pl.ds(k * 16, 16)] = zv


@jax.jit
def _sc_route(src, dst, s, d):
    ept = E // (NC * NS)

    def body(src_h, dst_h, s_h, d_h, sr_h, dr_h,
             s_v, d_v, src_v, dst_v, sro, dro):
        c = lax.axis_index("c")
        t = lax.axis_index("s")
        w = c * NS + t
        pltpu.sync_copy(s_h, s_v)
        pltpu.sync_copy(d_h, d_v)
        pltpu.sync_copy(src_h.at[pl.ds(w * ept, ept)], src_v)
        pltpu.sync_copy(dst_h.at[pl.ds(w * ept, ept)], dst_v)
        zero16 = jnp.zeros((16,), i32)
        npv = jnp.full((16,), NP_, i32)

        def it(k, _):
            sl = pl.ds(k * 16, 16)
            s16 = src_v[sl]
            d16 = dst_v[sl]
            sg = plsc.load_gather(s_v, [s16])
            dg = plsc.load_gather(d_v, [d16])
            off = jnp.where(sg + dg > 0.0, zero16, npv)
            sro[sl] = s16 + off
            dro[sl] = d16 + off
            return 0

        lax.fori_loop(0, ept // 16, it, 0)
        pltpu.sync_copy(sro, sr_h.at[pl.ds(w * ept, ept)])
        pltpu.sync_copy(dro, dr_h.at[pl.ds(w * ept, ept)])

    mesh = plsc.VectorSubcoreMesh(core_axis_name="c", subcore_axis_name="s")
    run = pl.kernel(
        body,
        out_type=[jax.ShapeDtypeStruct((E,), i32),
                  jax.ShapeDtypeStruct((E,), i32)],
        mesh=mesh,
        scratch_types=[
            pltpu.VMEM((NP_,), f32),
            pltpu.VMEM((NP_,), f32),
            pltpu.VMEM((ept,), i32),
            pltpu.VMEM((ept,), i32),
            pltpu.VMEM((ept,), i32),
            pltpu.VMEM((ept,), i32),
        ],
        compiler_params=pltpu.CompilerParams(needs_layout_passes=False, use_tc_tiling_on_sc=False),
    )
    return run(src, dst, s, d)


@functools.partial(jax.jit, static_argnames=("cols", "routed"))
def _sc_rows(table3, sidx2, didx2, *, cols, routed):
    B = 80
    NBUF = 3
    CH = 25
    arows = (2 * NP_) if routed else NP_
    ept = E // NS if routed else E // (NC * NS)
    nbatch = ept // B
    nchunk = nbatch // CH
    rpt = arows // NS
    ZR = 80

    def body(table_h, sidx_h, didx_h, out_h,
             sic, dic, r0, r1, r2, acc, gs0, gs1, gs2, ss0, ss1, ss2):
        c = lax.axis_index("c")
        t = lax.axis_index("s")
        rows_b = (r0, r1, r2)
        gsem = (gs0, gs1, gs2)
        ssem = (ss0, ss1, ss2)
        tab = table_h.at[c] if routed else table_h.at[0]
        w = t if routed else c * NS + t
        _zero_fill(r0, ZR, cols)
        base = t * rpt
        for j in range(rpt // ZR):
            pltpu.sync_copy(r0, acc.at[pl.ds(base + j * ZR, ZR)])
        plsc.subcore_barrier()

        brow0 = w * nbatch
        for ch in range(nchunk):
            pltpu.sync_copy(sidx_h.at[pl.ds(brow0 + ch * CH, CH)], sic)
            pltpu.sync_copy(didx_h.at[pl.ds(brow0 + ch * CH, CH)], dic)
            pltpu.async_copy(tab.at[sic.at[0]], r0, gs0)
            pltpu.async_copy(tab.at[sic.at[1]], r1, gs1)
            for j in range(CH):
                r = j % NBUF
                jn = j + 2
                if jn < CH:
                    rn = jn % NBUF
                    if j >= 1:
                        pltpu.make_async_copy(rows_b[(j - 1) % NBUF],
                                              acc.at[dic.at[j - 1]],
                                              ssem[(j - 1) % NBUF]).wait()
                    pltpu.async_copy(tab.at[sic.at[jn]], rows_b[rn], gsem[rn])
                pltpu.make_async_copy(tab.at[sic.at[j]], rows_b[r],
                                      gsem[r]).wait()
                pltpu.async_copy(rows_b[r], acc.at[dic.at[j]], ssem[r],
                                 add=True)
            for j in (CH - 3, CH - 2, CH - 1):
                pltpu.make_async_copy(rows_b[j % NBUF],
                                      acc.at[dic.at[j]],
                                      ssem[j % NBUF]).wait()
        plsc.subcore_barrier()
        pltpu.sync_copy(acc.at[pl.ds(base, rpt)],
                        out_h.at[pl.ds(c * arows + base, rpt)])

    mesh = plsc.VectorSubcoreMesh(core_axis_name="c", subcore_axis_name="s")
    run = pl.kernel(
        body,
        out_type=jax.ShapeDtypeStruct((2 * arows, cols), f32),
        mesh=mesh,
        scratch_types=[
            pltpu.VMEM((CH, B), i32),
            pltpu.VMEM((CH, B), i32),
            pltpu.VMEM((B, cols), f32),
            pltpu.VMEM((B, cols), f32),
            pltpu.VMEM((B, cols), f32),
            pltpu.VMEM_SHARED((arows, cols), f32),
            pltpu.SemaphoreType.DMA,
            pltpu.SemaphoreType.DMA,
            pltpu.SemaphoreType.DMA,
            pltpu.SemaphoreType.DMA,
            pltpu.SemaphoreType.DMA,
            pltpu.SemaphoreType.DMA,
        ],
        compiler_params=pltpu.CompilerParams(needs_layout_passes=False, use_tc_tiling_on_sc=False),
    )
    return run(table3, sidx2, didx2)


def _full(shape):
    return pl.BlockSpec(shape, lambda i: (0,) * len(shape))


def _prep(x, W0, b0, Wc, A):
    def body(x_r, w0_r, b0_r, wc_r, a_r, xp_r, sd_r, bm_r):
        y = jnp.dot(x_r[...], w0_r[...], preferred_element_type=f32) + b0_r[...]
        xp = jnp.dot(y, wc_r[...], preferred_element_type=f32)
        sd = jnp.dot(xp, a_r[...], preferred_element_type=f32)
        xp_r[...] = xp
        sd_r[...] = sd
        bm_r[...] = jnp.max(sd, axis=0, keepdims=True)[None]

    return pl.pallas_call(
        body,
        grid=(NBLK,),
        in_specs=[
            pl.BlockSpec((128, D), lambda i: (i, 0)),
            _full((D, H)), _full((1, H)), _full((H, H)), _full((H, 2)),
        ],
        out_specs=[
            pl.BlockSpec((128, H), lambda i: (i, 0)),
            pl.BlockSpec((128, 2), lambda i: (i, 0)),
            pl.BlockSpec((1, 1, 2), lambda i: (i, 0, 0)),
        ],
        out_shape=[
            jax.ShapeDtypeStruct((NP_, H), f32),
            jax.ShapeDtypeStruct((NP_, 2), f32),
            jax.ShapeDtypeStruct((NBLK, 1, 2), f32),
        ],
    )(x, W0, b0, Wc, A)


def _tables(xp, sd, K, cf, cols):
    ch = cf // 2

    def body(xp_r, sd_r, k_r, tab_r, dd_r):
        s = sd_r[...][:, 0:1]
        d = sd_r[...][:, 1:2]
        kv = k_r[...]
        p = jnp.exp(s - kv[0, 0])
        n = jnp.exp(0.2 * s - kv[0, 1])
        dd_r[...] = jnp.concatenate(
            [jnp.exp(d - kv[0, 2]), jnp.exp(0.2 * d - kv[0, 3])], axis=1)
        xpv = xp_r[...]
        pad = jnp.zeros((128, cols - ch - 1), f32)
        for c in range(2):
            xc = xpv[:, c * ch:(c + 1) * ch]
            tab_r[c, 0, :, :] = jnp.concatenate([xc * p, p, pad], axis=1)
            tab_r[c, 1, :, :] = jnp.concatenate([xc * n, n, pad], axis=1)

    return pl.pallas_call(
        body,
        grid=(NBLK,),
        in_specs=[
            pl.BlockSpec((128, cf), lambda i: (i, 0)),
            pl.BlockSpec((128, 2), lambda i: (i, 0)),
            _full((1, 4)),
        ],
        out_specs=[
            pl.BlockSpec((2, 2, 128, cols), lambda i: (0, 0, i, 0)),
            pl.BlockSpec((128, 2), lambda i: (i, 0)),
        ],
        out_shape=[
            jax.ShapeDtypeStruct((2, 2, NP_, cols), f32),
            jax.ShapeDtypeStruct((NP_, 2), f32),
        ],
    )(xp, sd, K)


def _combine(accd, dd, bias, cf, cols, act):
    ch = cf // 2

    def body(a0_r, a1_r, a2_r, a3_r, dd_r, b_r, y_r):
        dP = dd_r[...][:, 0:1]
        dN = dd_r[...][:, 1:2]
        halves = []
        for (ap, an) in ((a0_r, a1_r), (a2_r, a3_r)):
            apv = ap[...]
            anv = an[...]
            num = dP * apv[:, :ch] + dN * anv[:, :ch]
            den = dP * apv[:, ch:ch + 1] + dN * anv[:, ch:ch + 1] + 1e-16
            halves.append(num / den)
        y = jnp.concatenate(halves, axis=1) + b_r[...]
        if act == "relu":
            y = jnp.maximum(y, 0.0)
        y_r[...] = y

    specs = [pl.BlockSpec((128, cols), lambda i, q=q: (q * NBLK + i, 0))
             for q in range(4)]
    return pl.pallas_call(
        body,
        grid=(NBLK,),
        in_specs=specs + [
            pl.BlockSpec((128, 2), lambda i: (i, 0)),
            _full((1, cf)),
        ],
        out_specs=pl.BlockSpec((128, cf), lambda i: (i, 0)),
        out_shape=jax.ShapeDtypeStruct((NP_, cf), f32),
    )(accd, accd, accd, accd, dd, bias)


def _fuse2(accd, dd, bias, Wi, Ai, Ws, Wd, a_s1, a_d1):
    ch = H // 2

    def body(a0_r, a1_r, a2_r, a3_r, dd_r, b_r, wi_r, ai_r, ws_r, wd_r,
             as_r, ad_r, xpi_r, sdi_r, bmi_r, xps_r, sdj_r, bmj_r):
        dP = dd_r[...][:, 0:1]
        dN = dd_r[...][:, 1:2]
        halves = []
        for (ap, an) in ((a0_r, a1_r), (a2_r, a3_r)):
            apv = ap[...]
            anv = an[...]
            num = dP * apv[:, :ch] + dN * anv[:, :ch]
            den = dP * apv[:, ch:ch + 1] + dN * anv[:, ch:ch + 1] + 1e-16
            halves.append(num / den)
        y = jnp.concatenate(halves, axis=1) + b_r[...]
        y = jnp.maximum(y, 0.0)
        xe = jnp.where(y > 0, y, jnp.exp(jnp.minimum(y, 0.0)) - 1.0)
        xpi = jnp.dot(xe, wi_r[...], preferred_element_type=f32)
        sdi = jnp.dot(xpi, ai_r[...], preferred_element_type=f32)
        xpi_r[...] = xpi
        sdi_r[...] = sdi
        bmi_r[...] = jnp.max(sdi, axis=0, keepdims=True)[None]
        xps = jnp.dot(xe, ws_r[...], preferred_element_type=f32)
        xpd = jnp.dot(xe, wd_r[...], preferred_element_type=f32)
        xps_r[...] = xps
        sdj = jnp.concatenate(
            [jnp.dot(xps, as_r[...], preferred_element_type=f32),
             jnp.dot(xpd, ad_r[...], preferred_element_type=f32)], axis=1)
        sdj_r[...] = sdj
        bmj_r[...] = jnp.max(sdj, axis=0, keepdims=True)[None]

    specs = [pl.BlockSpec((128, 80), lambda i, q=q: (q * NBLK + i, 0))
             for q in range(4)]
    return pl.pallas_call(
        body,
        grid=(NBLK,),
        in_specs=specs + [
            pl.BlockSpec((128, 2), lambda i: (i, 0)),
            _full((1, H)), _full((H, HO)), _full((HO, 2)),
            _full((H, HO)), _full((H, HO)), _full((HO, 1)), _full((HO, 1)),
        ],
        out_specs=[
            pl.BlockSpec((128, HO), lambda i: (i, 0)),
            pl.BlockSpec((128, 2), lambda i: (i, 0)),
            pl.BlockSpec((1, 1, 2), lambda i: (i, 0, 0)),
            pl.BlockSpec((128, HO), lambda i: (i, 0)),
            pl.BlockSpec((128, 2), lambda i: (i, 0)),
            pl.BlockSpec((1, 1, 2), lambda i: (i, 0, 0)),
        ],
        out_shape=[
            jax.ShapeDtypeStruct((NP_, HO), f32),
            jax.ShapeDtypeStruct((NP_, 2), f32),
            jax.ShapeDtypeStruct((NBLK, 1, 2), f32),
            jax.ShapeDtypeStruct((NP_, HO), f32),
            jax.ShapeDtypeStruct((NP_, 2), f32),
            jax.ShapeDtypeStruct((NBLK, 1, 2), f32),
        ],
    )(accd, accd, accd, accd, dd, bias, Wi, Ai, Ws, Wd, a_s1, a_d1)


def _tables2(xp1, sd1, K1, xp2, sd2, K2):
    ch = HO // 2
    cols = 48

    def one(xp_r, sd_r, k_r, tab_r, dd_r):
        s = sd_r[...][:, 0:1]
        d = sd_r[...][:, 1:2]
        kv = k_r[...]
        p = jnp.exp(s - kv[0, 0])
        n = jnp.exp(0.2 * s - kv[0, 1])
        dd_r[...] = jnp.concatenate(
            [jnp.exp(d - kv[0, 2]), jnp.exp(0.2 * d - kv[0, 3])], axis=1)
        xpv = xp_r[...]
        pad = jnp.zeros((128, cols - ch - 1), f32)
        for c in range(2):
            xc = xpv[:, c * ch:(c + 1) * ch]
            tab_r[c, 0, :, :] = jnp.concatenate([xc * p, p, pad], axis=1)
            tab_r[c, 1, :, :] = jnp.concatenate([xc * n, n, pad], axis=1)

    def body(xp1_r, sd1_r, k1_r, xp2_r, sd2_r, k2_r,
             tab1_r, dd1_r, tab2_r, dd2_r):
        one(xp1_r, sd1_r, k1_r, tab1_r, dd1_r)
        one(xp2_r, sd2_r, k2_r, tab2_r, dd2_r)

    tspec = pl.BlockSpec((2, 2, 128, cols), lambda i: (0, 0, i, 0))
    vspec = pl.BlockSpec((128, 2), lambda i: (i, 0))
    xspec = pl.BlockSpec((128, HO), lambda i: (i, 0))
    tshape = jax.ShapeDtypeStruct((2, 2, NP_, cols), f32)
    vshape = jax.ShapeDtypeStruct((NP_, 2), f32)
    return pl.pallas_call(
        body,
        grid=(NBLK,),
        in_specs=[xspec, vspec, _full((1, 4)), xspec, vspec, _full((1, 4))],
        out_specs=[tspec, vspec, tspec, vspec],
        out_shape=[tshape, vshape, tshape, vshape],
    )(xp1, sd1, K1, xp2, sd2, K2)


def _sag_prep(rep, W12):
    def body(x_r, w_r, sg_r, tab_r):
        sg = jnp.dot(x_r[...], w_r[...], preferred_element_type=f32)
        sg_r[...] = sg
        tab_r[...] = jnp.concatenate(
            [sg[:, 1:2], jnp.zeros((128, 15), f32)], axis=1)

    return pl.pallas_call(
        body,
        grid=(NBLK,),
        in_specs=[pl.BlockSpec((128, H), lambda i: (i, 0)), _full((H, 2))],
        out_specs=[
            pl.BlockSpec((128, 2), lambda i: (i, 0)),
            pl.BlockSpec((128, 16), lambda i: (i, 0)),
        ],
        out_shape=[
            jax.ShapeDtypeStruct((NP_, 2), f32),
            jax.ShapeDtypeStruct((NP_, 16), f32),
        ],
    )(rep, W12)


def _pooling(rep, sg, aggd, batch2, sb):
    def body(rep_r, sg_r, agg_r, b2_r, sb_r, emb_r, wm_r):
        aggv = agg_r[...][:NP_, 0] + agg_r[...][NP_:, 0]
        sraw = sg_r[...][:, 0] + aggv + sb_r[0, 0]
        s2 = sraw.reshape(NBLK, 128)
        b2 = b2_r[...]
        neg = jnp.float32(-1e30)

        def mloop(g, mpick):
            mask = b2 == g
            mg = jnp.max(jnp.where(mask, s2, neg))
            mg = jnp.where(mg > -1e29, mg, 0.0)
            return mpick + jnp.where(mask, mg, 0.0)

        mpick = lax.fori_loop(0, G, mloop, jnp.zeros((NBLK, 128), f32))
        e2 = jnp.exp(s2 - mpick)

        def zloop(g, zpick):
            mask = b2 == g
            zg = jnp.sum(jnp.where(mask, e2, 0.0))
            return zpick + jnp.where(mask, zg, 0.0)

        zpick = lax.fori_loop(0, G, zloop, jnp.zeros((NBLK, 128), f32))
        score2 = e2 / (zpick + 1e-16)
        sflat = score2.reshape(1, NP_)
        bflat = b2.reshape(1, NP_)

        def wloop(g, _):
            wm_r[pl.ds(g, 1), :] = jnp.where(bflat == g, sflat, 0.0)
            return 0

        lax.fori_loop(0, G, wloop, 0)
        emb_r[...] = jnp.dot(wm_r[...], rep_r[...],
                             preferred_element_type=f32)

    def _f0(shape):
        return pl.BlockSpec(shape, lambda: (0,) * len(shape))

    return pl.pallas_call(
        body,
        in_specs=[
            _f0((NP_, H)), _f0((NP_, 2)), _f0((2 * NP_, 16)),
            _f0((NBLK, 128)), _f0((1, 1)),
        ],
        out_specs=_f0((G, H)),
        out_shape=jax.ShapeDtypeStruct((G, H), f32),
        scratch_shapes=[pltpu.VMEM((G, NP_), f32)],
    )(rep, sg, aggd, batch2, sb)


def _shifts(bmax_s, bmax_d):
    ks = jnp.max(bmax_s)
    kd = jnp.max(bmax_d)
    bc = 0.4 * (ks + kd)
    return jnp.stack([ks, 0.2 * ks + bc, kd, 0.2 * kd + bc]).reshape(1, 4)


def _pad_rows(x):
    return jnp.pad(x, ((0, NP_ - N), (0, 0)))


def kernel(h_x, t_x, h_edge_index, t_edge_index, b_edge_index, h_batch,
           t_batch, lin0_W, lin0_b, conv_W, conv_as, conv_ad, conv_b,
           intra_W, intra_as, intra_ad, intra_b, inter_Ws, inter_Wd,
           inter_as, inter_ad, inter_b, sag_W1, sag_W2, sag_b):
    hx = _pad_rows(h_x)
    tx = _pad_rows(t_x)
    h_src = h_edge_index[0].astype(i32)
    h_dst = h_edge_index[1].astype(i32)
    t_src = t_edge_index[0].astype(i32)
    t_dst = t_edge_index[1].astype(i32)
    b_src = b_edge_index[0].astype(i32)
    b_dst = b_edge_index[1].astype(i32)
    b0 = lin0_b.reshape(1, H)

    A_conv = jnp.stack([conv_as, conv_ad], axis=1)
    bias_conv = conv_b.reshape(1, H)

    def conv_stage(x, src, dst):
        xp, sd, bm = _prep(x, lin0_W, b0, conv_W, A_conv)
        K = _shifts(bm[:, 0, 0], bm[:, 0, 1])
        tab, dd = _tables(xp, sd, K, H, 80)
        sr, dr = _sc_route(src, dst, sd[:, 0], sd[:, 1])
        accd = _sc_rows(tab.reshape(2, 2 * NP_, 80), sr.reshape(E // 80, 80),
                        dr.reshape(E // 80, 80), cols=80, routed=True)
        return accd, dd

    accd_h, dd_h = conv_stage(hx, h_src, h_dst)
    accd_t, dd_t = conv_stage(tx, t_src, t_dst)

    A_intra = jnp.stack([intra_as, intra_ad], axis=1)
    bias_intra = intra_b.reshape(1, HO)
    bias_inter = inter_b.reshape(1, HO)
    as1 = inter_as.reshape(HO, 1)
    ad1 = inter_ad.reshape(HO, 1)

    (xpi_h, sdi_h, bmi_h, xps_h, sdj_h, bmj_h) = _fuse2(
        accd_h, dd_h, bias_conv, intra_W, A_intra, inter_Ws, inter_Wd,
        as1, ad1)
    (xpi_t, sdi_t, bmi_t, xps_t, sdj_t, bmj_t) = _fuse2(
        accd_t, dd_t, bias_conv, intra_W, A_intra, inter_Ws, inter_Wd,
        as1, ad1)

    Ki_h = _shifts(bmi_h[:, 0, 0], bmi_h[:, 0, 1])
    Ki_t = _shifts(bmi_t[:, 0, 0], bmi_t[:, 0, 1])
    ks_h = jnp.max(bmj_h[:, 0, 0]); kd_h = jnp.max(bmj_h[:, 0, 1])
    ks_t = jnp.max(bmj_t[:, 0, 0]); kd_t = jnp.max(bmj_t[:, 0, 1])
    Kj_h = jnp.stack([ks_h, 0.2 * ks_h + 0.4 * (ks_h + kd_t),
                      kd_h, 0.2 * kd_h + 0.4 * (ks_t + kd_h)]).reshape(1, 4)
    Kj_t = jnp.stack([ks_t, 0.2 * ks_t + 0.4 * (ks_t + kd_h),
                      kd_t, 0.2 * kd_t + 0.4 * (ks_h + kd_t)]).reshape(1, 4)

    tabi_h, ddi_h, tabj_h, ddj_h = _tables2(xpi_h, sdi_h, Ki_h,
                                            xps_h, sdj_h, Kj_h)
    tabi_t, ddi_t, tabj_t, ddj_t = _tables2(xpi_t, sdi_t, Ki_t,
                                            xps_t, sdj_t, Kj_t)

    def edge48(tab, src, dst, s_vec, d_vec, ddx, bias):
        sr, dr = _sc_route(src, dst, s_vec, d_vec)
        accd = _sc_rows(tab.reshape(2, 2 * NP_, 48), sr.reshape(E // 80, 80),
                        dr.reshape(E // 80, 80), cols=48, routed=True)
        return _combine(accd, ddx, bias, HO, 48, "none")

    h_intra = edge48(tabi_h, h_src, h_dst, sdi_h[:, 0], sdi_h[:, 1],
                     ddi_h, bias_intra)
    t_intra = edge48(tabi_t, t_src, t_dst, sdi_t[:, 0], sdi_t[:, 1],
                     ddi_t, bias_intra)
    t_inter = edge48(tabj_h, b_src, b_dst, sdj_h[:, 0], sdj_t[:, 1],
                     ddj_t, bias_inter)
    h_inter = edge48(tabj_t, b_dst, b_src, sdj_t[:, 0], sdj_h[:, 1],
                     ddj_h, bias_inter)

    h_rep_p = jnp.concatenate([h_intra, h_inter], axis=1)
    t_rep_p = jnp.concatenate([t_intra, t_inter], axis=1)

    W12 = jnp.concatenate([sag_W1, sag_W2], axis=1)
    sbv = sag_b.reshape(1, 1)

    def pool_stage(rep_p, src, dst, batch):
        sg, tab16 = _sag_prep(rep_p, W12)
        aggd = _sc_rows(tab16[None], src.reshape(E // 80, 80),
                        dst.reshape(E // 80, 80), cols=16, routed=False)
        b2 = jnp.pad(batch.astype(i32), (0, NP_ - N),
                     constant_values=G).reshape(NBLK, 128)
        return _pooling(rep_p, sg, aggd, b2, sbv)

    h_emb = pool_stage(h_rep_p, h_src, h_dst, h_batch)
    t_emb = pool_stage(t_rep_p, t_src, t_dst, t_batch)

    return (h_rep_p[:N], t_rep_p[:N], h_emb, t_emb)
